# baseline (speedup 1.0000x reference)
import functools

import jax
import jax.numpy as jnp
from jax import lax
from jax.experimental import pallas as pl
from jax.experimental.pallas import tpu as pltpu
from jax.experimental.pallas import tpu_sc as plsc

N_FIELDS = 26
BATCH = 16384
DIM = 36
NC = 2
NS = 16
NW = NC * NS
BPW = BATCH // NW
CH = 128
NCH = BPW // CH

_mesh = plsc.VectorSubcoreMesh(core_axis_name="c", subcore_axis_name="s")


@functools.partial(
    pl.kernel,
    mesh=_mesh,
    out_type=jax.ShapeDtypeStruct((BATCH * N_FIELDS, DIM), jnp.float32),
    scratch_types=[
        pltpu.VMEM((N_FIELDS, NCH, CH), jnp.int32),
        pltpu.VMEM((N_FIELDS, NCH, CH), jnp.int32),
        pltpu.VMEM((CH, DIM), jnp.float32),
        pltpu.SemaphoreType.DMA,
    ],
    compiler_params=pltpu.CompilerParams(use_tc_tiling_on_sc=False),
)
def _encode(idx_hbm, oidx_hbm, *rest):
    tables = rest[:N_FIELDS]
    out_hbm, idx_v, oidx_v, rows_v, sem = rest[N_FIELDS:]
    wid = lax.axis_index("s") * NC + lax.axis_index("c")
    pltpu.sync_copy(idx_hbm.at[wid], idx_v)
    pltpu.sync_copy(oidx_hbm.at[wid], oidx_v)
    for f in range(N_FIELDS):
        tbl = tables[f]
        for c in range(NCH):
            pltpu.async_copy(tbl.at[idx_v.at[f, c]], rows_v, sem).wait()
            pltpu.async_copy(rows_v, out_hbm.at[oidx_v.at[f, c]], sem).wait()


def kernel(x, table_0, table_1, table_2, table_3, table_4, table_5, table_6,
           table_7, table_8, table_9, table_10, table_11, table_12, table_13,
           table_14, table_15, table_16, table_17, table_18, table_19,
           table_20, table_21, table_22, table_23, table_24, table_25):
    tables = (table_0, table_1, table_2, table_3, table_4, table_5, table_6,
              table_7, table_8, table_9, table_10, table_11, table_12,
              table_13, table_14, table_15, table_16, table_17, table_18,
              table_19, table_20, table_21, table_22, table_23, table_24,
              table_25)
    xprep = x.reshape(NW, NCH, CH, N_FIELDS).transpose(0, 3, 1, 2)
    b = (jnp.arange(BATCH, dtype=jnp.int32)
         .reshape(NW, NCH, CH)[:, None, :, :])
    fcol = jnp.arange(N_FIELDS, dtype=jnp.int32)[None, :, None, None]
    oidx = b * N_FIELDS + fcol
    out = _encode(xprep, oidx, *tables)
    return out.reshape(BATCH, N_FIELDS * DIM)


# speedup vs baseline: 1.1037x; 1.1037x over previous
import functools

import jax
import jax.numpy as jnp
from jax import lax
from jax.experimental import pallas as pl
from jax.experimental.pallas import tpu as pltpu
from jax.experimental.pallas import tpu_sc as plsc

N_FIELDS = 26
BATCH = 16384
DIM = 36
NC = 2
NS = 16
NW = NC * NS
BPW = BATCH // NW
DIMP = 40
CH = 128
NCH = BPW // CH

_mesh = plsc.VectorSubcoreMesh(core_axis_name="c", subcore_axis_name="s")


@functools.partial(
    pl.kernel,
    mesh=_mesh,
    out_type=jax.ShapeDtypeStruct((BATCH * N_FIELDS, DIMP), jnp.float32),
    scratch_types=[
        pltpu.VMEM((N_FIELDS, NCH, CH), jnp.int32),
        pltpu.VMEM((N_FIELDS, NCH, CH), jnp.int32),
        pltpu.VMEM((CH, DIMP), jnp.float32),
        pltpu.SemaphoreType.DMA,
    ],
    compiler_params=pltpu.CompilerParams(use_tc_tiling_on_sc=False),
)
def _encode(idx_hbm, oidx_hbm, *rest):
    tables = rest[:N_FIELDS]
    out_hbm, idx_v, oidx_v, rows_v, sem = rest[N_FIELDS:]
    wid = lax.axis_index("s") * NC + lax.axis_index("c")
    pltpu.sync_copy(idx_hbm.at[wid], idx_v)
    pltpu.sync_copy(oidx_hbm.at[wid], oidx_v)
    for f in range(N_FIELDS):
        tbl = tables[f]
        for c in range(NCH):
            pltpu.async_copy(tbl.at[idx_v.at[f, c]], rows_v, sem).wait()
            pltpu.async_copy(rows_v, out_hbm.at[oidx_v.at[f, c]], sem).wait()


def kernel(x, table_0, table_1, table_2, table_3, table_4, table_5, table_6,
           table_7, table_8, table_9, table_10, table_11, table_12, table_13,
           table_14, table_15, table_16, table_17, table_18, table_19,
           table_20, table_21, table_22, table_23, table_24, table_25):
    tables = (table_0, table_1, table_2, table_3, table_4, table_5, table_6,
              table_7, table_8, table_9, table_10, table_11, table_12,
              table_13, table_14, table_15, table_16, table_17, table_18,
              table_19, table_20, table_21, table_22, table_23, table_24,
              table_25)
    tables = tuple(jnp.pad(t, ((0, 0), (0, DIMP - DIM))) for t in tables)
    xprep = x.reshape(NW, NCH, CH, N_FIELDS).transpose(0, 3, 1, 2)
    b = (jnp.arange(BATCH, dtype=jnp.int32)
         .reshape(NW, NCH, CH)[:, None, :, :])
    fcol = jnp.arange(N_FIELDS, dtype=jnp.int32)[None, :, None, None]
    oidx = b * N_FIELDS + fcol
    out = _encode(xprep, oidx, *tables)
    return out[:, :DIM].reshape(BATCH, N_FIELDS * DIM)


# speedup vs baseline: 7.0393x; 6.3777x over previous
import functools

import jax
import jax.numpy as jnp
from jax import lax
from jax.experimental import pallas as pl
from jax.experimental.pallas import tpu as pltpu
from jax.experimental.pallas import tpu_sc as plsc

N_FIELDS = 26
BATCH = 16384
DIM = 36
VOCAB = 100000
NC = 2
NS = 16
NW = NC * NS
NU = N_FIELDS * DIM
HB = BATCH // 2
LANES = 16

_mesh = plsc.VectorSubcoreMesh(core_axis_name="c", subcore_axis_name="s")


@functools.partial(
    pl.kernel,
    mesh=_mesh,
    out_type=jax.ShapeDtypeStruct((NU, BATCH), jnp.float32),
    scratch_types=[
        pltpu.VMEM((VOCAB,), jnp.float32),
        pltpu.VMEM((HB,), jnp.int32),
        pltpu.VMEM((HB,), jnp.float32),
    ],
    compiler_params=pltpu.CompilerParams(needs_layout_passes=False),
)
def _encode(xt_hbm, *rest):
    tables = rest[:N_FIELDS]
    out_hbm, row_v, idx_v, och_v = rest[N_FIELDS:]
    wid = lax.axis_index("s") * NC + lax.axis_index("c")
    lo_u = (wid * NU) // NW
    hi_u = ((wid + 1) * NU) // NW

    for f in range(N_FIELDS):
        tbl = tables[f]
        dlo = jnp.clip(lo_u - f * DIM, 0, DIM)
        dhi = jnp.clip(hi_u - f * DIM, 0, DIM)

        def dbody(d, _, f=f, tbl=tbl):
            pltpu.sync_copy(tbl.at[d], row_v)
            for h in range(2):
                pltpu.sync_copy(xt_hbm.at[f, pl.ds(h * HB, HB)], idx_v)

                def gbody(c, _):
                    iv = idx_v[pl.ds(c * LANES, LANES)]
                    och_v[pl.ds(c * LANES, LANES)] = plsc.load_gather(
                        row_v, [iv])
                    return 0

                lax.fori_loop(0, HB // LANES, gbody, 0)
                pltpu.sync_copy(
                    och_v, out_hbm.at[f * DIM + d, pl.ds(h * HB, HB)])
            return 0

        lax.fori_loop(dlo, dhi, dbody, 0)


def kernel(x, table_0, table_1, table_2, table_3, table_4, table_5, table_6,
           table_7, table_8, table_9, table_10, table_11, table_12, table_13,
           table_14, table_15, table_16, table_17, table_18, table_19,
           table_20, table_21, table_22, table_23, table_24, table_25):
    tables = (table_0, table_1, table_2, table_3, table_4, table_5, table_6,
              table_7, table_8, table_9, table_10, table_11, table_12,
              table_13, table_14, table_15, table_16, table_17, table_18,
              table_19, table_20, table_21, table_22, table_23, table_24,
              table_25)
    out_t = _encode(x.T, *(t.T for t in tables))
    return out_t.T.reshape(BATCH, N_FIELDS * DIM)


# speedup vs baseline: 9.5747x; 1.3602x over previous
import functools

import jax
import jax.numpy as jnp
from jax import lax
from jax.experimental import pallas as pl
from jax.experimental.pallas import tpu as pltpu
from jax.experimental.pallas import tpu_sc as plsc

N_FIELDS = 26
BATCH = 16384
DIM = 36
VOCAB = 100000
NC = 2
NS = 16
NW = NC * NS
NU = N_FIELDS * DIM
HB = BATCH // 2
LANES = 16

_mesh = plsc.VectorSubcoreMesh(core_axis_name="c", subcore_axis_name="s")


@functools.partial(
    pl.kernel,
    mesh=_mesh,
    out_type=jax.ShapeDtypeStruct((NU, BATCH), jnp.float32),
    scratch_types=[
        pltpu.VMEM((VOCAB,), jnp.float32),
        pltpu.VMEM((HB,), jnp.int32),
        pltpu.VMEM((HB,), jnp.int32),
        pltpu.VMEM((HB,), jnp.float32),
    ],
    compiler_params=pltpu.CompilerParams(needs_layout_passes=False),
)
def _encode(xt_hbm, *rest):
    tables = rest[:N_FIELDS]
    out_hbm, row_v, idxa_v, idxb_v, och_v = rest[N_FIELDS:]
    wid = lax.axis_index("s") * NC + lax.axis_index("c")
    lo_u = (wid * NU) // NW
    hi_u = ((wid + 1) * NU) // NW
    UNROLL = 8

    for f in range(N_FIELDS):
        tbl = tables[f]
        dlo = jnp.clip(lo_u - f * DIM, 0, DIM)
        dhi = jnp.clip(hi_u - f * DIM, 0, DIM)

        @pl.when(dhi > dlo)
        def _(f=f):
            pltpu.sync_copy(xt_hbm.at[f, pl.ds(0, HB)], idxa_v)
            pltpu.sync_copy(xt_hbm.at[f, pl.ds(HB, HB)], idxb_v)

        def dbody(d, _, f=f, tbl=tbl):
            pltpu.sync_copy(tbl.at[d], row_v)
            for h, idx_v in ((0, idxa_v), (1, idxb_v)):

                def gbody(c, _, idx_v=idx_v):
                    base = c * (LANES * UNROLL)
                    for u in range(UNROLL):
                        iv = idx_v[pl.ds(base + u * LANES, LANES)]
                        och_v[pl.ds(base + u * LANES, LANES)] = (
                            plsc.load_gather(row_v, [iv]))
                    return 0

                lax.fori_loop(0, HB // (LANES * UNROLL), gbody, 0)
                pltpu.sync_copy(
                    och_v, out_hbm.at[f * DIM + d, pl.ds(h * HB, HB)])
            return 0

        lax.fori_loop(dlo, dhi, dbody, 0)


def kernel(x, table_0, table_1, table_2, table_3, table_4, table_5, table_6,
           table_7, table_8, table_9, table_10, table_11, table_12, table_13,
           table_14, table_15, table_16, table_17, table_18, table_19,
           table_20, table_21, table_22, table_23, table_24, table_25):
    tables = (table_0, table_1, table_2, table_3, table_4, table_5, table_6,
              table_7, table_8, table_9, table_10, table_11, table_12,
              table_13, table_14, table_15, table_16, table_17, table_18,
              table_19, table_20, table_21, table_22, table_23, table_24,
              table_25)
    out_t = _encode(x.T, *(t.T for t in tables))
    return out_t.T.reshape(BATCH, N_FIELDS * DIM)
